# baseline (speedup 1.0000x reference)
import functools

import jax
import jax.numpy as jnp
from jax import lax
from jax.experimental import pallas as pl
from jax.experimental.pallas import tpu as pltpu
from jax.experimental.pallas import tpu_sc as plsc

NP = 10240
EP = 327680
RB = 1024
GRID = NP // RB
DOP = 128
NG = 64

_HIGH = lax.Precision.HIGHEST


def _dot(a, b, dims):
    return lax.dot_general(a, b, dims, precision=_HIGH,
                           preferred_element_type=jnp.float32)


def _pre_body(degp_ref, x_ref, dis_ref, g0_ref):
    t = degp_ref[...]
    deg = t[0, :, 0] + t[1, :, 0] + 1.0
    dis = lax.rsqrt(deg)
    dis_ref[...] = jnp.broadcast_to(dis[:, None], (RB, 128))
    g0_ref[...] = x_ref[...] * dis[:, None]


def _layer1_body(acc_ref, g0_ref, dis_ref, w_ref, b_ref, g1_ref):
    dis = dis_ref[:, 0:1]
    p = dis * (acc_ref[0] + acc_ref[1] + g0_ref[...])
    h = jnp.maximum(_dot(p, w_ref[...], (((1,), (0,)), ((), ()))) +
                    b_ref[...], 0.0)
    g = h * dis
    g1_ref[0] = g[:, :128]
    g1_ref[1] = g[:, 128:]


def _mid_body(acc_ref, g_ref, dis_ref, w_ref, b_ref, out_ref):
    dis = dis_ref[:, 0:1]
    p0 = dis * (acc_ref[0] + g_ref[0])
    p1 = dis * (acc_ref[1] + g_ref[1])
    h = (_dot(p0, w_ref[0:128, :], (((1,), (0,)), ((), ()))) +
         _dot(p1, w_ref[128:256, :], (((1,), (0,)), ((), ()))) + b_ref[...])
    h = jnp.maximum(h, 0.0)
    g = h * dis
    out_ref[0] = g[:, :128]
    out_ref[1] = g[:, 128:]


def _layer3_body(acc_ref, g_ref, dis_ref, w_ref, b_ref, wout_ref, r3_ref):
    dis = dis_ref[:, 0:1]
    p0 = dis * (acc_ref[0] + g_ref[0])
    p1 = dis * (acc_ref[1] + g_ref[1])
    h = (_dot(p0, w_ref[0:128, :], (((1,), (0,)), ((), ()))) +
         _dot(p1, w_ref[128:256, :], (((1,), (0,)), ((), ()))) + b_ref[...])
    h = jnp.maximum(h, 0.0)
    q = _dot(h, wout_ref[...], (((1,), (0,)), ((), ())))
    r3_ref[...] = q * dis


def _final_body(acc_ref, r3_ref, dis_ref, bout_ref, batch_ref, out_ref,
                s_acc, c_acc):
    i = pl.program_id(0)

    @pl.when(i == 0)
    def _():
        s_acc[...] = jnp.zeros_like(s_acc)
        c_acc[...] = jnp.zeros_like(c_acc)

    dis = dis_ref[:, 0:1]
    y = dis * (acc_ref[0] + acc_ref[1] + r3_ref[...]) + bout_ref[...]
    bt = batch_ref[0, 0]
    onehot = (bt[:, None] == lax.broadcasted_iota(jnp.int32, (RB, NG), 1)
              ).astype(jnp.float32)
    s_acc[...] += _dot(onehot, y, (((0,), (0,)), ((), ())))
    c_acc[...] += jnp.sum(onehot, axis=0, keepdims=True)

    @pl.when(i == GRID - 1)
    def _():
        pooled = s_acc[...] / jnp.maximum(c_acc[...], 1.0).T
        valid = lax.broadcasted_iota(jnp.int32, (NG, DOP), 1) < 40
        neg = jnp.float32(-1e30)
        masked = jnp.where(valid, pooled, neg)
        m = jnp.max(masked, axis=1, keepdims=True)
        e = jnp.where(valid, jnp.exp(masked - m), 0.0)
        lse = jnp.log(jnp.sum(e, axis=1, keepdims=True))
        out_ref[...] = jnp.where(valid, masked - m - lse, 0.0)


def _tc_pre(deg_parts, x_pad):
    return pl.pallas_call(
        _pre_body,
        grid=(GRID,),
        in_specs=[
            pl.BlockSpec((2, RB, 128), lambda i: (0, i, 0)),
            pl.BlockSpec((RB, 128), lambda i: (i, 0)),
        ],
        out_specs=[
            pl.BlockSpec((RB, 128), lambda i: (i, 0)),
            pl.BlockSpec((RB, 128), lambda i: (i, 0)),
        ],
        out_shape=[
            jax.ShapeDtypeStruct((NP, 128), jnp.float32),
            jax.ShapeDtypeStruct((NP, 128), jnp.float32),
        ],
    )(deg_parts, x_pad)


def _tc_layer1(acc, g0, dis_b, W_in, b_in):
    return pl.pallas_call(
        _layer1_body,
        grid=(GRID,),
        in_specs=[
            pl.BlockSpec((2, RB, 128), lambda i: (0, i, 0)),
            pl.BlockSpec((RB, 128), lambda i: (i, 0)),
            pl.BlockSpec((RB, 128), lambda i: (i, 0)),
            pl.BlockSpec((128, 256), lambda i: (0, 0)),
            pl.BlockSpec((256,), lambda i: (0,)),
        ],
        out_specs=pl.BlockSpec((2, RB, 128), lambda i: (0, i, 0)),
        out_shape=jax.ShapeDtypeStruct((2, NP, 128), jnp.float32),
    )(acc, g0, dis_b, W_in, b_in)


def _tc_mid(acc, g, dis_b, W, b):
    return pl.pallas_call(
        _mid_body,
        grid=(GRID,),
        in_specs=[
            pl.BlockSpec((2, RB, 128), lambda i: (0, i, 0)),
            pl.BlockSpec((2, RB, 128), lambda i: (0, i, 0)),
            pl.BlockSpec((RB, 128), lambda i: (i, 0)),
            pl.BlockSpec((256, 256), lambda i: (0, 0)),
            pl.BlockSpec((256,), lambda i: (0,)),
        ],
        out_specs=pl.BlockSpec((2, RB, 128), lambda i: (0, i, 0)),
        out_shape=jax.ShapeDtypeStruct((2, NP, 128), jnp.float32),
    )(acc, g, dis_b, W, b)


def _tc_layer3(acc, g, dis_b, W, b, W_out_pad):
    return pl.pallas_call(
        _layer3_body,
        grid=(GRID,),
        in_specs=[
            pl.BlockSpec((2, RB, 128), lambda i: (0, i, 0)),
            pl.BlockSpec((2, RB, 128), lambda i: (0, i, 0)),
            pl.BlockSpec((RB, 128), lambda i: (i, 0)),
            pl.BlockSpec((256, 256), lambda i: (0, 0)),
            pl.BlockSpec((256,), lambda i: (0,)),
            pl.BlockSpec((256, DOP), lambda i: (0, 0)),
        ],
        out_specs=pl.BlockSpec((RB, DOP), lambda i: (i, 0)),
        out_shape=jax.ShapeDtypeStruct((NP, DOP), jnp.float32),
    )(acc, g, dis_b, W, b, W_out_pad)


def _tc_final(acc, r3, dis_b, b_out_pad, batch2):
    return pl.pallas_call(
        _final_body,
        grid=(GRID,),
        in_specs=[
            pl.BlockSpec((2, RB, DOP), lambda i: (0, i, 0)),
            pl.BlockSpec((RB, DOP), lambda i: (i, 0)),
            pl.BlockSpec((RB, 128), lambda i: (i, 0)),
            pl.BlockSpec((DOP,), lambda i: (0,)),
            pl.BlockSpec((1, 1, RB), lambda i: (i, 0, 0)),
        ],
        out_specs=pl.BlockSpec((NG, DOP), lambda i: (0, 0)),
        out_shape=jax.ShapeDtypeStruct((NG, DOP), jnp.float32),
        scratch_shapes=[
            pltpu.VMEM((NG, DOP), jnp.float32),
            pltpu.VMEM((1, NG), jnp.float32),
        ],
    )(acc, r3, dis_b, b_out_pad, batch2)


NC, NS = 2, 16
K = 64
RPW = NP // NS

_sc_mesh = plsc.VectorSubcoreMesh(core_axis_name="c", subcore_axis_name="s",
                                  num_cores=NC, num_subcores=NS)


def _zero_acc_and_barrier(rows_v, acc, s, fc):
    def zrow(k, _):
        for j in range(fc // 16):
            rows_v[k, pl.ds(16 * j, 16)] = jnp.zeros((16,), jnp.float32)
        return 0

    lax.fori_loop(0, K, zrow, 0)

    def zcp(j, _):
        pltpu.sync_copy(rows_v, acc.at[pl.ds(s * RPW + j * K, K)])
        return 0

    lax.fori_loop(0, RPW // K, zcp, 0)
    plsc.subcore_barrier()


def _writeback(acc, out_h, c, s):
    plsc.subcore_barrier()

    def wb(j, _):
        r0 = s * RPW + j * K
        pltpu.sync_copy(acc.at[pl.ds(r0, K)], out_h.at[pl.ds(c * NP + r0, K)])
        return 0

    lax.fori_loop(0, RPW // K, wb, 0)


GB = 8


NBUF = 4


def _make_sc_propagate(fc, feat_split):
    rpw = ((EP // NS) if feat_split else (EP // (NC * NS))) // K
    ngroups = rpw // GB

    def body(tbl_h, src_h, dst_h, w_h, out_h,
             idxs_b, idxd_b, w_b, r0_, r1_, r2_, r3_, acc,
             g0_, g1_, g2_, g3_, s0_, s1_, s2_, s3_):
        c = lax.axis_index("c")
        s = lax.axis_index("s")
        base = (s if feat_split else s * NC + c) * rpw
        rows = (r0_, r1_, r2_, r3_)
        gsem = (g0_, g1_, g2_, g3_)
        ssem = (s0_, s1_, s2_, s3_)
        _zero_acc_and_barrier(rows[0], acc, s, fc)

        def group(gi, _):
            r0 = base + gi * GB
            pltpu.sync_copy(src_h.at[pl.ds(r0, GB)], idxs_b)
            pltpu.sync_copy(dst_h.at[pl.ds(r0, GB)], idxd_b)
            pltpu.sync_copy(w_h.at[pl.ds(r0, GB)], w_b)
            if feat_split:
                toff = c * NP

                def adj(j, _):
                    for u in range(K // 16):
                        sl = pl.ds(16 * u, 16)
                        idxs_b[j, sl] = idxs_b[j, sl] + toff
                    return 0

                lax.fori_loop(0, GB, adj, 0)
            gpend = [None] * GB
            wpend = [None] * GB
            for j in range(min(2, GB)):
                gpend[j] = pltpu.async_copy(tbl_h.at[idxs_b.at[j]],
                                            rows[j % NBUF], gsem[j % NBUF])
            for j in range(GB):
                b = j % NBUF
                gpend[j].wait()

                def sgroup(g, _, _j=j, _b=b):
                    wv = w_b[_j, pl.ds(16 * g, 16)]
                    for l in range(16):
                        wk = wv[l]
                        k = 16 * g + l
                        for u in range(fc // 16):
                            sl = pl.ds(16 * u, 16)
                            rows[_b][k, sl] = rows[_b][k, sl] * wk
                    return 0

                lax.fori_loop(0, K // 16, sgroup, 0)
                wpend[j] = pltpu.async_copy(rows[b], acc.at[idxd_b.at[j]],
                                            ssem[b], add=True)
                if j + 2 < GB:
                    b2 = (j + 2) % NBUF
                    if j + 2 >= NBUF:
                        wpend[j + 2 - NBUF].wait()
                    gpend[j + 2] = pltpu.async_copy(tbl_h.at[idxs_b.at[j + 2]],
                                                    rows[b2], gsem[b2])
            for j in range(max(0, GB - NBUF), GB):
                wpend[j].wait()
            return 0

        lax.fori_loop(0, ngroups, group, 0)
        _writeback(acc, out_h, c, s)

    return pl.kernel(
        body,
        out_type=jax.ShapeDtypeStruct((2 * NP, fc), jnp.float32),
        mesh=_sc_mesh,
        scratch_types=[
            pltpu.VMEM((GB, K), jnp.int32),
            pltpu.VMEM((GB, K), jnp.int32),
            pltpu.VMEM((GB, K), jnp.float32),
            pltpu.VMEM((K, fc), jnp.float32),
            pltpu.VMEM((K, fc), jnp.float32),
            pltpu.VMEM((K, fc), jnp.float32),
            pltpu.VMEM((K, fc), jnp.float32),
            pltpu.VMEM_SHARED((NP, fc), jnp.float32),
            pltpu.SemaphoreType.DMA,
            pltpu.SemaphoreType.DMA,
            pltpu.SemaphoreType.DMA,
            pltpu.SemaphoreType.DMA,
            pltpu.SemaphoreType.DMA,
            pltpu.SemaphoreType.DMA,
            pltpu.SemaphoreType.DMA,
            pltpu.SemaphoreType.DMA,
        ],
    )


def _make_sc_deg():
    fc = 128
    nloc = EP // (NC * NS)
    nchunks = nloc // K

    def body(dst_h, w_h, out_h, idxd_v, w_v, rows_v, acc):
        c = lax.axis_index("c")
        s = lax.axis_index("s")
        base = (s * NC + c) * nloc
        _zero_acc_and_barrier(rows_v, acc, s, fc)
        ones = jnp.full((16,), 1.0, jnp.float32)

        def chunk(i, _):
            eb = base + i * K
            pltpu.sync_copy(dst_h.at[pl.ds(eb, K)], idxd_v)
            pltpu.sync_copy(w_h.at[pl.ds(eb, K)], w_v)

            def sgroup(g, _):
                wv = w_v[pl.ds(16 * g, 16)]
                for l in range(16):
                    rows_v[16 * g + l, pl.ds(0, 16)] = ones * wv[l]
                return 0

            lax.fori_loop(0, K // 16, sgroup, 0)
            pltpu.sync_copy(rows_v, acc.at[idxd_v], add=True)
            return 0

        lax.fori_loop(0, nchunks, chunk, 0)
        _writeback(acc, out_h, c, s)

    return pl.kernel(
        body,
        out_type=jax.ShapeDtypeStruct((2 * NP, fc), jnp.float32),
        mesh=_sc_mesh,
        scratch_types=[
            pltpu.VMEM((K,), jnp.int32),
            pltpu.VMEM((K,), jnp.float32),
            pltpu.VMEM((K, fc), jnp.float32),
            pltpu.VMEM_SHARED((NP, fc), jnp.float32),
        ],
    )


_sc_deg = _make_sc_deg()
_sc_prop128_edge = _make_sc_propagate(128, feat_split=False)
_sc_prop128_feat = _make_sc_propagate(128, feat_split=True)


def kernel(x, edge_index, batch, edge_weight,
           W_in, b_in, W_h1, b_h1, W_h2, b_h2, W_out, b_out):
    n0, e0 = x.shape[0], edge_weight.shape[0]
    pad_idx = jnp.arange(EP - e0, dtype=jnp.int32) % NP
    src = jnp.concatenate([edge_index[0], pad_idx])
    dst = jnp.concatenate([edge_index[1], pad_idx])
    w = jnp.pad(edge_weight, (0, EP - e0))
    x_pad = jnp.pad(x, ((0, NP - n0), (0, 0)))
    batch2 = jnp.pad(batch, (0, NP - n0),
                     constant_values=NG).reshape(GRID, 1, RB)
    W_out_pad = jnp.pad(W_out, ((0, 0), (0, DOP - W_out.shape[1])))
    b_out_pad = jnp.pad(b_out, (0, DOP - b_out.shape[0]))

    src2 = src.reshape(EP // K, K)
    dst2 = dst.reshape(EP // K, K)
    w2 = w.reshape(EP // K, K)

    deg_parts = _sc_deg(dst, w).reshape(2, NP, 128)
    dis_b, g0 = _tc_pre(deg_parts, x_pad)

    acc1 = _sc_prop128_edge(g0, src2, dst2, w2).reshape(2, NP, 128)
    g1 = _tc_layer1(acc1, g0, dis_b, W_in, b_in)

    acc2 = _sc_prop128_feat(g1.reshape(2 * NP, 128), src2, dst2, w2)
    g2 = _tc_mid(acc2.reshape(2, NP, 128), g1, dis_b, W_h1, b_h1)

    acc3 = _sc_prop128_feat(g2.reshape(2 * NP, 128), src2, dst2, w2)
    r3 = _tc_layer3(acc3.reshape(2, NP, 128), g2, dis_b, W_h2, b_h2, W_out_pad)

    acc4 = _sc_prop128_edge(r3, src2, dst2, w2).reshape(2, NP, DOP)
    out = _tc_final(acc4, r3, dis_b, b_out_pad, batch2)

    return out[:, :40]


# speedup vs baseline: 1.0079x; 1.0079x over previous
import functools

import jax
import jax.numpy as jnp
from jax import lax
from jax.experimental import pallas as pl
from jax.experimental.pallas import tpu as pltpu
from jax.experimental.pallas import tpu_sc as plsc

NP = 10240
EP = 327680
RB = 1024
GRID = NP // RB
DOP = 128
NG = 64

_HIGH = lax.Precision.HIGHEST


def _dot(a, b, dims):
    return lax.dot_general(a, b, dims, precision=_HIGH,
                           preferred_element_type=jnp.float32)


def _pre_body(degp_ref, x_ref, dis_ref, g0_ref):
    t = degp_ref[...]
    deg = t[0, :, 0] + t[1, :, 0] + 1.0
    dis = lax.rsqrt(deg)
    dis_ref[...] = jnp.broadcast_to(dis[:, None], (RB, 128))
    g0_ref[...] = x_ref[...] * dis[:, None]


def _layer1_body(acc_ref, g0_ref, dis_ref, w_ref, b_ref, g1_ref):
    dis = dis_ref[:, 0:1]
    p = dis * (acc_ref[0] + acc_ref[1] + g0_ref[...])
    h = jnp.maximum(_dot(p, w_ref[...], (((1,), (0,)), ((), ()))) +
                    b_ref[...], 0.0)
    g = h * dis
    g1_ref[0] = g[:, :128]
    g1_ref[1] = g[:, 128:]


def _mid_body(acc_ref, g_ref, dis_ref, w_ref, b_ref, out_ref):
    dis = dis_ref[:, 0:1]
    p0 = dis * (acc_ref[0] + g_ref[0])
    p1 = dis * (acc_ref[1] + g_ref[1])
    h = (_dot(p0, w_ref[0:128, :], (((1,), (0,)), ((), ()))) +
         _dot(p1, w_ref[128:256, :], (((1,), (0,)), ((), ()))) + b_ref[...])
    h = jnp.maximum(h, 0.0)
    g = h * dis
    out_ref[0] = g[:, :128]
    out_ref[1] = g[:, 128:]


def _layer3_body(acc_ref, g_ref, dis_ref, w_ref, b_ref, wout_ref, r3_ref):
    dis = dis_ref[:, 0:1]
    p0 = dis * (acc_ref[0] + g_ref[0])
    p1 = dis * (acc_ref[1] + g_ref[1])
    h = (_dot(p0, w_ref[0:128, :], (((1,), (0,)), ((), ()))) +
         _dot(p1, w_ref[128:256, :], (((1,), (0,)), ((), ()))) + b_ref[...])
    h = jnp.maximum(h, 0.0)
    q = _dot(h, wout_ref[...], (((1,), (0,)), ((), ())))
    r3_ref[...] = q * dis


def _final_body(acc_ref, r3_ref, dis_ref, bout_ref, batch_ref, out_ref,
                s_acc, c_acc):
    i = pl.program_id(0)

    @pl.when(i == 0)
    def _():
        s_acc[...] = jnp.zeros_like(s_acc)
        c_acc[...] = jnp.zeros_like(c_acc)

    dis = dis_ref[:, 0:1]
    y = dis * (acc_ref[0] + acc_ref[1] + r3_ref[...]) + bout_ref[...]
    bt = batch_ref[0, 0]
    onehot = (bt[:, None] == lax.broadcasted_iota(jnp.int32, (RB, NG), 1)
              ).astype(jnp.float32)
    s_acc[...] += _dot(onehot, y, (((0,), (0,)), ((), ())))
    c_acc[...] += jnp.sum(onehot, axis=0, keepdims=True)

    @pl.when(i == GRID - 1)
    def _():
        pooled = s_acc[...] / jnp.maximum(c_acc[...], 1.0).T
        valid = lax.broadcasted_iota(jnp.int32, (NG, DOP), 1) < 40
        neg = jnp.float32(-1e30)
        masked = jnp.where(valid, pooled, neg)
        m = jnp.max(masked, axis=1, keepdims=True)
        e = jnp.where(valid, jnp.exp(masked - m), 0.0)
        lse = jnp.log(jnp.sum(e, axis=1, keepdims=True))
        out_ref[...] = jnp.where(valid, masked - m - lse, 0.0)


def _tc_pre(deg_parts, x_pad):
    return pl.pallas_call(
        _pre_body,
        grid=(GRID,),
        in_specs=[
            pl.BlockSpec((2, RB, 128), lambda i: (0, i, 0)),
            pl.BlockSpec((RB, 128), lambda i: (i, 0)),
        ],
        out_specs=[
            pl.BlockSpec((RB, 128), lambda i: (i, 0)),
            pl.BlockSpec((RB, 128), lambda i: (i, 0)),
        ],
        out_shape=[
            jax.ShapeDtypeStruct((NP, 128), jnp.float32),
            jax.ShapeDtypeStruct((NP, 128), jnp.float32),
        ],
    )(deg_parts, x_pad)


def _tc_layer1(acc, g0, dis_b, W_in, b_in):
    return pl.pallas_call(
        _layer1_body,
        grid=(GRID,),
        in_specs=[
            pl.BlockSpec((2, RB, 128), lambda i: (0, i, 0)),
            pl.BlockSpec((RB, 128), lambda i: (i, 0)),
            pl.BlockSpec((RB, 128), lambda i: (i, 0)),
            pl.BlockSpec((128, 256), lambda i: (0, 0)),
            pl.BlockSpec((256,), lambda i: (0,)),
        ],
        out_specs=pl.BlockSpec((2, RB, 128), lambda i: (0, i, 0)),
        out_shape=jax.ShapeDtypeStruct((2, NP, 128), jnp.float32),
    )(acc, g0, dis_b, W_in, b_in)


def _tc_mid(acc, g, dis_b, W, b):
    return pl.pallas_call(
        _mid_body,
        grid=(GRID,),
        in_specs=[
            pl.BlockSpec((2, RB, 128), lambda i: (0, i, 0)),
            pl.BlockSpec((2, RB, 128), lambda i: (0, i, 0)),
            pl.BlockSpec((RB, 128), lambda i: (i, 0)),
            pl.BlockSpec((256, 256), lambda i: (0, 0)),
            pl.BlockSpec((256,), lambda i: (0,)),
        ],
        out_specs=pl.BlockSpec((2, RB, 128), lambda i: (0, i, 0)),
        out_shape=jax.ShapeDtypeStruct((2, NP, 128), jnp.float32),
    )(acc, g, dis_b, W, b)


def _tc_layer3(acc, g, dis_b, W, b, W_out_pad):
    return pl.pallas_call(
        _layer3_body,
        grid=(GRID,),
        in_specs=[
            pl.BlockSpec((2, RB, 128), lambda i: (0, i, 0)),
            pl.BlockSpec((2, RB, 128), lambda i: (0, i, 0)),
            pl.BlockSpec((RB, 128), lambda i: (i, 0)),
            pl.BlockSpec((256, 256), lambda i: (0, 0)),
            pl.BlockSpec((256,), lambda i: (0,)),
            pl.BlockSpec((256, DOP), lambda i: (0, 0)),
        ],
        out_specs=pl.BlockSpec((RB, DOP), lambda i: (i, 0)),
        out_shape=jax.ShapeDtypeStruct((NP, DOP), jnp.float32),
    )(acc, g, dis_b, W, b, W_out_pad)


def _tc_final(acc, r3, dis_b, b_out_pad, batch2):
    return pl.pallas_call(
        _final_body,
        grid=(GRID,),
        in_specs=[
            pl.BlockSpec((2, RB, DOP), lambda i: (0, i, 0)),
            pl.BlockSpec((RB, DOP), lambda i: (i, 0)),
            pl.BlockSpec((RB, 128), lambda i: (i, 0)),
            pl.BlockSpec((DOP,), lambda i: (0,)),
            pl.BlockSpec((1, 1, RB), lambda i: (i, 0, 0)),
        ],
        out_specs=pl.BlockSpec((NG, DOP), lambda i: (0, 0)),
        out_shape=jax.ShapeDtypeStruct((NG, DOP), jnp.float32),
        scratch_shapes=[
            pltpu.VMEM((NG, DOP), jnp.float32),
            pltpu.VMEM((1, NG), jnp.float32),
        ],
    )(acc, r3, dis_b, b_out_pad, batch2)


NC, NS = 2, 16
K = 128
RPW = NP // NS

_sc_mesh = plsc.VectorSubcoreMesh(core_axis_name="c", subcore_axis_name="s",
                                  num_cores=NC, num_subcores=NS)


def _zero_acc_and_barrier(rows_v, acc, s, fc):
    def zrow(k, _):
        for j in range(fc // 16):
            rows_v[k, pl.ds(16 * j, 16)] = jnp.zeros((16,), jnp.float32)
        return 0

    lax.fori_loop(0, K, zrow, 0)

    def zcp(j, _):
        pltpu.sync_copy(rows_v, acc.at[pl.ds(s * RPW + j * K, K)])
        return 0

    lax.fori_loop(0, RPW // K, zcp, 0)
    plsc.subcore_barrier()


def _writeback(acc, out_h, c, s):
    plsc.subcore_barrier()

    def wb(j, _):
        r0 = s * RPW + j * K
        pltpu.sync_copy(acc.at[pl.ds(r0, K)], out_h.at[pl.ds(c * NP + r0, K)])
        return 0

    lax.fori_loop(0, RPW // K, wb, 0)


GB = 8


def _make_sc_propagate(fc, feat_split):
    rpw = ((EP // NS) if feat_split else (EP // (NC * NS))) // K
    ngroups = rpw // GB

    def body(tbl_h, src_h, dst_h, w_h, out_h,
             idxs_b, idxd_b, w_b, r0_, r1_, acc,
             g0_, g1_, s0_, s1_):
        c = lax.axis_index("c")
        s = lax.axis_index("s")
        base = (s if feat_split else s * NC + c) * rpw
        rows = (r0_, r1_)
        gsem = (g0_, g1_)
        ssem = (s0_, s1_)
        _zero_acc_and_barrier(rows[0], acc, s, fc)

        def group(gi, _):
            r0 = base + gi * GB
            pltpu.sync_copy(src_h.at[pl.ds(r0, GB)], idxs_b)
            pltpu.sync_copy(dst_h.at[pl.ds(r0, GB)], idxd_b)
            pltpu.sync_copy(w_h.at[pl.ds(r0, GB)], w_b)
            if feat_split:
                toff = c * NP

                def adj(j, _):
                    for u in range(K // 16):
                        sl = pl.ds(16 * u, 16)
                        idxs_b[j, sl] = idxs_b[j, sl] + toff
                    return 0

                lax.fori_loop(0, GB, adj, 0)
            gpend = [None] * GB
            wpend = [None] * GB
            gpend[0] = pltpu.async_copy(tbl_h.at[idxs_b.at[0]], rows[0],
                                        gsem[0])
            for j in range(GB):
                b = j % 2
                gpend[j].wait()

                def sgroup(g, _, _j=j, _b=b):
                    wv = w_b[_j, pl.ds(16 * g, 16)]
                    for l in range(16):
                        wk = wv[l]
                        k = 16 * g + l
                        for u in range(fc // 16):
                            sl = pl.ds(16 * u, 16)
                            rows[_b][k, sl] = rows[_b][k, sl] * wk
                    return 0

                lax.fori_loop(0, K // 16, sgroup, 0)
                wpend[j] = pltpu.async_copy(rows[b], acc.at[idxd_b.at[j]],
                                            ssem[b], add=True)
                if j + 1 < GB:
                    if j >= 1:
                        wpend[j - 1].wait()
                    gpend[j + 1] = pltpu.async_copy(tbl_h.at[idxs_b.at[j + 1]],
                                                    rows[1 - b], gsem[1 - b])
            for j in range(max(0, GB - 2), GB):
                wpend[j].wait()
            return 0

        lax.fori_loop(0, ngroups, group, 0)
        _writeback(acc, out_h, c, s)

    return pl.kernel(
        body,
        out_type=jax.ShapeDtypeStruct((2 * NP, fc), jnp.float32),
        mesh=_sc_mesh,
        scratch_types=[
            pltpu.VMEM((GB, K), jnp.int32),
            pltpu.VMEM((GB, K), jnp.int32),
            pltpu.VMEM((GB, K), jnp.float32),
            pltpu.VMEM((K, fc), jnp.float32),
            pltpu.VMEM((K, fc), jnp.float32),
            pltpu.VMEM_SHARED((NP, fc), jnp.float32),
            pltpu.SemaphoreType.DMA,
            pltpu.SemaphoreType.DMA,
            pltpu.SemaphoreType.DMA,
            pltpu.SemaphoreType.DMA,
        ],
    )


def _make_sc_deg():
    fc = 128
    nloc = EP // (NC * NS)
    nchunks = nloc // K

    def body(dst_h, w_h, out_h, idxd_v, w_v, rows_v, acc):
        c = lax.axis_index("c")
        s = lax.axis_index("s")
        base = (s * NC + c) * nloc
        _zero_acc_and_barrier(rows_v, acc, s, fc)
        ones = jnp.full((16,), 1.0, jnp.float32)

        def chunk(i, _):
            eb = base + i * K
            pltpu.sync_copy(dst_h.at[pl.ds(eb, K)], idxd_v)
            pltpu.sync_copy(w_h.at[pl.ds(eb, K)], w_v)

            def sgroup(g, _):
                wv = w_v[pl.ds(16 * g, 16)]
                for l in range(16):
                    rows_v[16 * g + l, pl.ds(0, 16)] = ones * wv[l]
                return 0

            lax.fori_loop(0, K // 16, sgroup, 0)
            pltpu.sync_copy(rows_v, acc.at[idxd_v], add=True)
            return 0

        lax.fori_loop(0, nchunks, chunk, 0)
        _writeback(acc, out_h, c, s)

    return pl.kernel(
        body,
        out_type=jax.ShapeDtypeStruct((2 * NP, fc), jnp.float32),
        mesh=_sc_mesh,
        scratch_types=[
            pltpu.VMEM((K,), jnp.int32),
            pltpu.VMEM((K,), jnp.float32),
            pltpu.VMEM((K, fc), jnp.float32),
            pltpu.VMEM_SHARED((NP, fc), jnp.float32),
        ],
    )


_sc_deg = _make_sc_deg()
_sc_prop128_edge = _make_sc_propagate(128, feat_split=False)
_sc_prop128_feat = _make_sc_propagate(128, feat_split=True)


def kernel(x, edge_index, batch, edge_weight,
           W_in, b_in, W_h1, b_h1, W_h2, b_h2, W_out, b_out):
    n0, e0 = x.shape[0], edge_weight.shape[0]
    pad_idx = jnp.arange(EP - e0, dtype=jnp.int32) % NP
    src = jnp.concatenate([edge_index[0], pad_idx])
    dst = jnp.concatenate([edge_index[1], pad_idx])
    w = jnp.pad(edge_weight, (0, EP - e0))
    x_pad = jnp.pad(x, ((0, NP - n0), (0, 0)))
    batch2 = jnp.pad(batch, (0, NP - n0),
                     constant_values=NG).reshape(GRID, 1, RB)
    W_out_pad = jnp.pad(W_out, ((0, 0), (0, DOP - W_out.shape[1])))
    b_out_pad = jnp.pad(b_out, (0, DOP - b_out.shape[0]))

    src2 = src.reshape(EP // K, K)
    dst2 = dst.reshape(EP // K, K)
    w2 = w.reshape(EP // K, K)

    deg_parts = _sc_deg(dst, w).reshape(2, NP, 128)
    dis_b, g0 = _tc_pre(deg_parts, x_pad)

    acc1 = _sc_prop128_edge(g0, src2, dst2, w2).reshape(2, NP, 128)
    g1 = _tc_layer1(acc1, g0, dis_b, W_in, b_in)

    acc2 = _sc_prop128_feat(g1.reshape(2 * NP, 128), src2, dst2, w2)
    g2 = _tc_mid(acc2.reshape(2, NP, 128), g1, dis_b, W_h1, b_h1)

    acc3 = _sc_prop128_feat(g2.reshape(2 * NP, 128), src2, dst2, w2)
    r3 = _tc_layer3(acc3.reshape(2, NP, 128), g2, dis_b, W_h2, b_h2, W_out_pad)

    acc4 = _sc_prop128_edge(r3, src2, dst2, w2).reshape(2, NP, DOP)
    out = _tc_final(acc4, r3, dis_b, b_out_pad, batch2)

    return out[:, :40]


# speedup vs baseline: 1.1677x; 1.1585x over previous
import functools

import jax
import jax.numpy as jnp
from jax import lax
from jax.experimental import pallas as pl
from jax.experimental.pallas import tpu as pltpu
from jax.experimental.pallas import tpu_sc as plsc

NP = 10240
EP = 327680
RB = 1024
GRID = NP // RB
DOP = 128
NG = 64

_HIGH = lax.Precision.HIGHEST


def _dot(a, b, dims):
    return lax.dot_general(a, b, dims, precision=_HIGH,
                           preferred_element_type=jnp.float32)


def _pre_body(degp_ref, x_ref, dis_ref, g0_ref):
    t = degp_ref[...]
    deg = t[0, :, 0] + t[1, :, 0] + 1.0
    dis = lax.rsqrt(deg)
    dis_ref[...] = jnp.broadcast_to(dis[:, None], (RB, 128))
    g0_ref[...] = x_ref[...] * dis[:, None]


def _layer1_body(acc_ref, g0_ref, dis_ref, w_ref, b_ref, g1_ref):
    dis = dis_ref[:, 0:1]
    p = dis * (acc_ref[0] + acc_ref[1] + g0_ref[...])
    h = jnp.maximum(_dot(p, w_ref[...], (((1,), (0,)), ((), ()))) +
                    b_ref[...], 0.0)
    g = h * dis
    g1_ref[0] = g[:, :128]
    g1_ref[1] = g[:, 128:]


def _mid_body(acc_ref, g_ref, dis_ref, w_ref, b_ref, out_ref):
    dis = dis_ref[:, 0:1]
    p0 = dis * (acc_ref[0] + g_ref[0])
    p1 = dis * (acc_ref[1] + g_ref[1])
    h = (_dot(p0, w_ref[0:128, :], (((1,), (0,)), ((), ()))) +
         _dot(p1, w_ref[128:256, :], (((1,), (0,)), ((), ()))) + b_ref[...])
    h = jnp.maximum(h, 0.0)
    g = h * dis
    out_ref[0] = g[:, :128]
    out_ref[1] = g[:, 128:]


def _layer3_body(acc_ref, g_ref, dis_ref, w_ref, b_ref, wout_ref, r3_ref):
    dis = dis_ref[:, 0:1]
    p0 = dis * (acc_ref[0] + g_ref[0])
    p1 = dis * (acc_ref[1] + g_ref[1])
    h = (_dot(p0, w_ref[0:128, :], (((1,), (0,)), ((), ()))) +
         _dot(p1, w_ref[128:256, :], (((1,), (0,)), ((), ()))) + b_ref[...])
    h = jnp.maximum(h, 0.0)
    q = _dot(h, wout_ref[...], (((1,), (0,)), ((), ())))
    r3_ref[...] = q * dis


def _final_body(acc_ref, r3_ref, dis_ref, bout_ref, batch_ref, out_ref,
                s_acc, c_acc):
    i = pl.program_id(0)

    @pl.when(i == 0)
    def _():
        s_acc[...] = jnp.zeros_like(s_acc)
        c_acc[...] = jnp.zeros_like(c_acc)

    dis = dis_ref[:, 0:1]
    y = dis * (acc_ref[0] + acc_ref[1] + r3_ref[...]) + bout_ref[...]
    bt = batch_ref[0, 0]
    onehot = (bt[:, None] == lax.broadcasted_iota(jnp.int32, (RB, NG), 1)
              ).astype(jnp.float32)
    s_acc[...] += _dot(onehot, y, (((0,), (0,)), ((), ())))
    c_acc[...] += jnp.sum(onehot, axis=0, keepdims=True)

    @pl.when(i == GRID - 1)
    def _():
        pooled = s_acc[...] / jnp.maximum(c_acc[...], 1.0).T
        valid = lax.broadcasted_iota(jnp.int32, (NG, DOP), 1) < 40
        neg = jnp.float32(-1e30)
        masked = jnp.where(valid, pooled, neg)
        m = jnp.max(masked, axis=1, keepdims=True)
        e = jnp.where(valid, jnp.exp(masked - m), 0.0)
        lse = jnp.log(jnp.sum(e, axis=1, keepdims=True))
        out_ref[...] = jnp.where(valid, masked - m - lse, 0.0)


def _tc_pre(deg_parts, x_pad):
    return pl.pallas_call(
        _pre_body,
        grid=(GRID,),
        in_specs=[
            pl.BlockSpec((2, RB, 128), lambda i: (0, i, 0)),
            pl.BlockSpec((RB, 128), lambda i: (i, 0)),
        ],
        out_specs=[
            pl.BlockSpec((RB, 128), lambda i: (i, 0)),
            pl.BlockSpec((RB, 128), lambda i: (i, 0)),
        ],
        out_shape=[
            jax.ShapeDtypeStruct((NP, 128), jnp.float32),
            jax.ShapeDtypeStruct((NP, 128), jnp.float32),
        ],
    )(deg_parts, x_pad)


def _tc_layer1(acc, g0, dis_b, W_in, b_in):
    return pl.pallas_call(
        _layer1_body,
        grid=(GRID,),
        in_specs=[
            pl.BlockSpec((2, RB, 128), lambda i: (0, i, 0)),
            pl.BlockSpec((RB, 128), lambda i: (i, 0)),
            pl.BlockSpec((RB, 128), lambda i: (i, 0)),
            pl.BlockSpec((128, 256), lambda i: (0, 0)),
            pl.BlockSpec((256,), lambda i: (0,)),
        ],
        out_specs=pl.BlockSpec((2, RB, 128), lambda i: (0, i, 0)),
        out_shape=jax.ShapeDtypeStruct((2, NP, 128), jnp.float32),
    )(acc, g0, dis_b, W_in, b_in)


def _tc_mid(acc, g, dis_b, W, b):
    return pl.pallas_call(
        _mid_body,
        grid=(GRID,),
        in_specs=[
            pl.BlockSpec((2, RB, 128), lambda i: (0, i, 0)),
            pl.BlockSpec((2, RB, 128), lambda i: (0, i, 0)),
            pl.BlockSpec((RB, 128), lambda i: (i, 0)),
            pl.BlockSpec((256, 256), lambda i: (0, 0)),
            pl.BlockSpec((256,), lambda i: (0,)),
        ],
        out_specs=pl.BlockSpec((2, RB, 128), lambda i: (0, i, 0)),
        out_shape=jax.ShapeDtypeStruct((2, NP, 128), jnp.float32),
    )(acc, g, dis_b, W, b)


def _tc_layer3(acc, g, dis_b, W, b, W_out_pad):
    return pl.pallas_call(
        _layer3_body,
        grid=(GRID,),
        in_specs=[
            pl.BlockSpec((2, RB, 128), lambda i: (0, i, 0)),
            pl.BlockSpec((2, RB, 128), lambda i: (0, i, 0)),
            pl.BlockSpec((RB, 128), lambda i: (i, 0)),
            pl.BlockSpec((256, 256), lambda i: (0, 0)),
            pl.BlockSpec((256,), lambda i: (0,)),
            pl.BlockSpec((256, DOP), lambda i: (0, 0)),
        ],
        out_specs=pl.BlockSpec((RB, DOP), lambda i: (i, 0)),
        out_shape=jax.ShapeDtypeStruct((NP, DOP), jnp.float32),
    )(acc, g, dis_b, W, b, W_out_pad)


def _tc_final(acc, r3, dis_b, b_out_pad, batch2):
    return pl.pallas_call(
        _final_body,
        grid=(GRID,),
        in_specs=[
            pl.BlockSpec((2, RB, DOP), lambda i: (0, i, 0)),
            pl.BlockSpec((RB, DOP), lambda i: (i, 0)),
            pl.BlockSpec((RB, 128), lambda i: (i, 0)),
            pl.BlockSpec((DOP,), lambda i: (0,)),
            pl.BlockSpec((1, 1, RB), lambda i: (i, 0, 0)),
        ],
        out_specs=pl.BlockSpec((NG, DOP), lambda i: (0, 0)),
        out_shape=jax.ShapeDtypeStruct((NG, DOP), jnp.float32),
        scratch_shapes=[
            pltpu.VMEM((NG, DOP), jnp.float32),
            pltpu.VMEM((1, NG), jnp.float32),
        ],
    )(acc, r3, dis_b, b_out_pad, batch2)


NC, NS = 2, 16
K = 128
RPW = NP // NS

_sc_mesh = plsc.VectorSubcoreMesh(core_axis_name="c", subcore_axis_name="s",
                                  num_cores=NC, num_subcores=NS)


def _zero_acc_and_barrier(rows_v, acc, s, fc):
    def zrow(k, _):
        for j in range(fc // 16):
            rows_v[k, pl.ds(16 * j, 16)] = jnp.zeros((16,), jnp.float32)
        return 0

    lax.fori_loop(0, K, zrow, 0)

    def zcp(j, _):
        pltpu.sync_copy(rows_v, acc.at[pl.ds(s * RPW + j * K, K)])
        return 0

    lax.fori_loop(0, RPW // K, zcp, 0)
    plsc.subcore_barrier()


def _writeback(acc, out_h, c, s):
    plsc.subcore_barrier()

    def wb(j, _):
        r0 = s * RPW + j * K
        pltpu.sync_copy(acc.at[pl.ds(r0, K)], out_h.at[pl.ds(c * NP + r0, K)])
        return 0

    lax.fori_loop(0, RPW // K, wb, 0)


GB = 8


def _make_sc_propagate(fc, feat_split):
    rpw = ((EP // NS) if feat_split else (EP // (NC * NS))) // K
    ngroups = rpw // GB

    def body(tbl_h, src_h, dst_h, w_h, out_h,
             idxs_b, idxd_b, w_b, r0_, r1_, acc,
             g0_, g1_, s0_, s1_):
        c = lax.axis_index("c")
        s = lax.axis_index("s")
        base = (s if feat_split else s * NC + c) * rpw
        rows = (r0_, r1_)
        gsem = (g0_, g1_)
        ssem = (s0_, s1_)
        _zero_acc_and_barrier(rows[0], acc, s, fc)

        def group(gi, _):
            r0 = base + gi * GB
            pltpu.sync_copy(src_h.at[pl.ds(r0, GB)], idxs_b)
            pltpu.sync_copy(dst_h.at[pl.ds(r0, GB)], idxd_b)
            pltpu.sync_copy(w_h.at[pl.ds(r0, GB)], w_b)
            if feat_split:
                toff = c * NP

                def adj(j, _):
                    for u in range(K // 16):
                        sl = pl.ds(16 * u, 16)
                        idxs_b[j, sl] = idxs_b[j, sl] + toff
                    return 0

                lax.fori_loop(0, GB, adj, 0)
            gpend = [None] * GB
            gpend[0] = pltpu.async_copy(tbl_h.at[idxs_b.at[0]], rows[0],
                                        gsem[0])
            for j in range(GB):
                b = j % 2
                if j + 1 < GB:
                    gpend[j + 1] = pltpu.async_copy(tbl_h.at[idxs_b.at[j + 1]],
                                                    rows[1 - b], gsem[1 - b])
                gpend[j].wait()

                def sgroup(g, _, _j=j, _b=b):
                    wv = w_b[_j, pl.ds(16 * g, 16)]
                    for l in range(16):
                        wk = wv[l]
                        k = 16 * g + l
                        for u in range(fc // 16):
                            sl = pl.ds(16 * u, 16)
                            rows[_b][k, sl] = rows[_b][k, sl] * wk
                    return 0

                lax.fori_loop(0, K // 16, sgroup, 0)
                pltpu.sync_copy(rows[b], acc.at[idxd_b.at[j]], add=True)
            return 0

        lax.fori_loop(0, ngroups, group, 0)
        _writeback(acc, out_h, c, s)

    return pl.kernel(
        body,
        out_type=jax.ShapeDtypeStruct((2 * NP, fc), jnp.float32),
        mesh=_sc_mesh,
        scratch_types=[
            pltpu.VMEM((GB, K), jnp.int32),
            pltpu.VMEM((GB, K), jnp.int32),
            pltpu.VMEM((GB, K), jnp.float32),
            pltpu.VMEM((K, fc), jnp.float32),
            pltpu.VMEM((K, fc), jnp.float32),
            pltpu.VMEM_SHARED((NP, fc), jnp.float32),
            pltpu.SemaphoreType.DMA,
            pltpu.SemaphoreType.DMA,
            pltpu.SemaphoreType.DMA,
            pltpu.SemaphoreType.DMA,
        ],
    )


def _make_sc_deg():
    fc = 128
    nloc = EP // (NC * NS)
    nchunks = nloc // K

    def body(dst_h, w_h, out_h, idxd_v, w_v, rows_v, acc):
        c = lax.axis_index("c")
        s = lax.axis_index("s")
        base = (s * NC + c) * nloc
        _zero_acc_and_barrier(rows_v, acc, s, fc)
        ones = jnp.full((16,), 1.0, jnp.float32)

        def chunk(i, _):
            eb = base + i * K
            pltpu.sync_copy(dst_h.at[pl.ds(eb, K)], idxd_v)
            pltpu.sync_copy(w_h.at[pl.ds(eb, K)], w_v)

            def sgroup(g, _):
                wv = w_v[pl.ds(16 * g, 16)]
                for l in range(16):
                    rows_v[16 * g + l, pl.ds(0, 16)] = ones * wv[l]
                return 0

            lax.fori_loop(0, K // 16, sgroup, 0)
            pltpu.sync_copy(rows_v, acc.at[idxd_v], add=True)
            return 0

        lax.fori_loop(0, nchunks, chunk, 0)
        _writeback(acc, out_h, c, s)

    return pl.kernel(
        body,
        out_type=jax.ShapeDtypeStruct((2 * NP, fc), jnp.float32),
        mesh=_sc_mesh,
        scratch_types=[
            pltpu.VMEM((K,), jnp.int32),
            pltpu.VMEM((K,), jnp.float32),
            pltpu.VMEM((K, fc), jnp.float32),
            pltpu.VMEM_SHARED((NP, fc), jnp.float32),
        ],
    )


_sc_deg = _make_sc_deg()
_sc_prop128_edge = _make_sc_propagate(128, feat_split=False)
_sc_prop128_feat = _make_sc_propagate(128, feat_split=True)


def kernel(x, edge_index, batch, edge_weight,
           W_in, b_in, W_h1, b_h1, W_h2, b_h2, W_out, b_out):
    n0, e0 = x.shape[0], edge_weight.shape[0]
    pad_idx = jnp.arange(EP - e0, dtype=jnp.int32) % NP
    src = jnp.concatenate([edge_index[0], pad_idx])
    dst = jnp.concatenate([edge_index[1], pad_idx])
    w = jnp.pad(edge_weight, (0, EP - e0))
    x_pad = jnp.pad(x, ((0, NP - n0), (0, 0)))
    batch2 = jnp.pad(batch, (0, NP - n0),
                     constant_values=NG).reshape(GRID, 1, RB)
    W_out_pad = jnp.pad(W_out, ((0, 0), (0, DOP - W_out.shape[1])))
    b_out_pad = jnp.pad(b_out, (0, DOP - b_out.shape[0]))

    src2 = src.reshape(EP // K, K)
    dst2 = dst.reshape(EP // K, K)
    w2 = w.reshape(EP // K, K)

    deg_parts = _sc_deg(dst, w).reshape(2, NP, 128)
    dis_b, g0 = _tc_pre(deg_parts, x_pad)

    acc1 = _sc_prop128_edge(g0, src2, dst2, w2).reshape(2, NP, 128)
    g1 = _tc_layer1(acc1, g0, dis_b, W_in, b_in)

    acc2 = _sc_prop128_feat(g1.reshape(2 * NP, 128), src2, dst2, w2)
    g2 = _tc_mid(acc2.reshape(2, NP, 128), g1, dis_b, W_h1, b_h1)

    acc3 = _sc_prop128_feat(g2.reshape(2 * NP, 128), src2, dst2, w2)
    r3 = _tc_layer3(acc3.reshape(2, NP, 128), g2, dis_b, W_h2, b_h2, W_out_pad)

    acc4 = _sc_prop128_edge(r3, src2, dst2, w2).reshape(2, NP, DOP)
    out = _tc_final(acc4, r3, dis_b, b_out_pad, batch2)

    return out[:, :40]


# speedup vs baseline: 1.1719x; 1.0037x over previous
import functools

import jax
import jax.numpy as jnp
from jax import lax
from jax.experimental import pallas as pl
from jax.experimental.pallas import tpu as pltpu
from jax.experimental.pallas import tpu_sc as plsc

NP = 10240
EP = 327680
RB = 1024
GRID = NP // RB
DOP = 128
NG = 64

_HIGH = lax.Precision.HIGHEST


def _dot(a, b, dims):
    return lax.dot_general(a, b, dims, precision=_HIGH,
                           preferred_element_type=jnp.float32)


def _pre_body(degp_ref, x_ref, dis_ref, g0_ref):
    t = degp_ref[...]
    deg = t[0, :, 0] + t[1, :, 0] + 1.0
    dis = lax.rsqrt(deg)
    dis_ref[...] = jnp.broadcast_to(dis[:, None], (RB, 128))
    g0_ref[...] = x_ref[...] * dis[:, None]


def _layer1_body(acc_ref, g0_ref, dis_ref, w_ref, b_ref, g1_ref):
    dis = dis_ref[:, 0:1]
    p = dis * (acc_ref[0] + acc_ref[1] + g0_ref[...])
    h = jnp.maximum(_dot(p, w_ref[...], (((1,), (0,)), ((), ()))) +
                    b_ref[...], 0.0)
    g = h * dis
    g1_ref[0] = g[:, :128]
    g1_ref[1] = g[:, 128:]


def _mid_body(acc_ref, g_ref, dis_ref, w_ref, b_ref, out_ref):
    dis = dis_ref[:, 0:1]
    p0 = dis * (acc_ref[0] + g_ref[0])
    p1 = dis * (acc_ref[1] + g_ref[1])
    h = (_dot(p0, w_ref[0:128, :], (((1,), (0,)), ((), ()))) +
         _dot(p1, w_ref[128:256, :], (((1,), (0,)), ((), ()))) + b_ref[...])
    h = jnp.maximum(h, 0.0)
    g = h * dis
    out_ref[0] = g[:, :128]
    out_ref[1] = g[:, 128:]


def _layer3_body(acc_ref, g_ref, dis_ref, w_ref, b_ref, wout_ref, r3_ref):
    dis = dis_ref[:, 0:1]
    p0 = dis * (acc_ref[0] + g_ref[0])
    p1 = dis * (acc_ref[1] + g_ref[1])
    h = (_dot(p0, w_ref[0:128, :], (((1,), (0,)), ((), ()))) +
         _dot(p1, w_ref[128:256, :], (((1,), (0,)), ((), ()))) + b_ref[...])
    h = jnp.maximum(h, 0.0)
    q = _dot(h, wout_ref[...], (((1,), (0,)), ((), ())))
    r3_ref[...] = q * dis


def _final_body(acc_ref, r3_ref, dis_ref, bout_ref, batch_ref, out_ref,
                s_acc, c_acc):
    i = pl.program_id(0)

    @pl.when(i == 0)
    def _():
        s_acc[...] = jnp.zeros_like(s_acc)
        c_acc[...] = jnp.zeros_like(c_acc)

    dis = dis_ref[:, 0:1]
    y = dis * (acc_ref[0] + acc_ref[1] + r3_ref[...]) + bout_ref[...]
    bt = batch_ref[0, 0]
    onehot = (bt[:, None] == lax.broadcasted_iota(jnp.int32, (RB, NG), 1)
              ).astype(jnp.float32)
    s_acc[...] += _dot(onehot, y, (((0,), (0,)), ((), ())))
    c_acc[...] += jnp.sum(onehot, axis=0, keepdims=True)

    @pl.when(i == GRID - 1)
    def _():
        pooled = s_acc[...] / jnp.maximum(c_acc[...], 1.0).T
        valid = lax.broadcasted_iota(jnp.int32, (NG, DOP), 1) < 40
        neg = jnp.float32(-1e30)
        masked = jnp.where(valid, pooled, neg)
        m = jnp.max(masked, axis=1, keepdims=True)
        e = jnp.where(valid, jnp.exp(masked - m), 0.0)
        lse = jnp.log(jnp.sum(e, axis=1, keepdims=True))
        out_ref[...] = jnp.where(valid, masked - m - lse, 0.0)


def _tc_pre(deg_parts, x_pad):
    return pl.pallas_call(
        _pre_body,
        grid=(GRID,),
        in_specs=[
            pl.BlockSpec((2, RB, 128), lambda i: (0, i, 0)),
            pl.BlockSpec((RB, 128), lambda i: (i, 0)),
        ],
        out_specs=[
            pl.BlockSpec((RB, 128), lambda i: (i, 0)),
            pl.BlockSpec((RB, 128), lambda i: (i, 0)),
        ],
        out_shape=[
            jax.ShapeDtypeStruct((NP, 128), jnp.float32),
            jax.ShapeDtypeStruct((NP, 128), jnp.float32),
        ],
    )(deg_parts, x_pad)


def _tc_layer1(acc, g0, dis_b, W_in, b_in):
    return pl.pallas_call(
        _layer1_body,
        grid=(GRID,),
        in_specs=[
            pl.BlockSpec((2, RB, 128), lambda i: (0, i, 0)),
            pl.BlockSpec((RB, 128), lambda i: (i, 0)),
            pl.BlockSpec((RB, 128), lambda i: (i, 0)),
            pl.BlockSpec((128, 256), lambda i: (0, 0)),
            pl.BlockSpec((256,), lambda i: (0,)),
        ],
        out_specs=pl.BlockSpec((2, RB, 128), lambda i: (0, i, 0)),
        out_shape=jax.ShapeDtypeStruct((2, NP, 128), jnp.float32),
    )(acc, g0, dis_b, W_in, b_in)


def _tc_mid(acc, g, dis_b, W, b):
    return pl.pallas_call(
        _mid_body,
        grid=(GRID,),
        in_specs=[
            pl.BlockSpec((2, RB, 128), lambda i: (0, i, 0)),
            pl.BlockSpec((2, RB, 128), lambda i: (0, i, 0)),
            pl.BlockSpec((RB, 128), lambda i: (i, 0)),
            pl.BlockSpec((256, 256), lambda i: (0, 0)),
            pl.BlockSpec((256,), lambda i: (0,)),
        ],
        out_specs=pl.BlockSpec((2, RB, 128), lambda i: (0, i, 0)),
        out_shape=jax.ShapeDtypeStruct((2, NP, 128), jnp.float32),
    )(acc, g, dis_b, W, b)


def _tc_layer3(acc, g, dis_b, W, b, W_out_pad):
    return pl.pallas_call(
        _layer3_body,
        grid=(GRID,),
        in_specs=[
            pl.BlockSpec((2, RB, 128), lambda i: (0, i, 0)),
            pl.BlockSpec((2, RB, 128), lambda i: (0, i, 0)),
            pl.BlockSpec((RB, 128), lambda i: (i, 0)),
            pl.BlockSpec((256, 256), lambda i: (0, 0)),
            pl.BlockSpec((256,), lambda i: (0,)),
            pl.BlockSpec((256, DOP), lambda i: (0, 0)),
        ],
        out_specs=pl.BlockSpec((RB, DOP), lambda i: (i, 0)),
        out_shape=jax.ShapeDtypeStruct((NP, DOP), jnp.float32),
    )(acc, g, dis_b, W, b, W_out_pad)


def _tc_final(acc, r3, dis_b, b_out_pad, batch2):
    return pl.pallas_call(
        _final_body,
        grid=(GRID,),
        in_specs=[
            pl.BlockSpec((2, RB, DOP), lambda i: (0, i, 0)),
            pl.BlockSpec((RB, DOP), lambda i: (i, 0)),
            pl.BlockSpec((RB, 128), lambda i: (i, 0)),
            pl.BlockSpec((DOP,), lambda i: (0,)),
            pl.BlockSpec((1, 1, RB), lambda i: (i, 0, 0)),
        ],
        out_specs=pl.BlockSpec((NG, DOP), lambda i: (0, 0)),
        out_shape=jax.ShapeDtypeStruct((NG, DOP), jnp.float32),
        scratch_shapes=[
            pltpu.VMEM((NG, DOP), jnp.float32),
            pltpu.VMEM((1, NG), jnp.float32),
        ],
    )(acc, r3, dis_b, b_out_pad, batch2)


NC, NS = 2, 16
K = 128
RPW = NP // NS

_sc_mesh = plsc.VectorSubcoreMesh(core_axis_name="c", subcore_axis_name="s",
                                  num_cores=NC, num_subcores=NS)


def _zero_acc_and_barrier(rows_v, acc, s, fc):
    def zrow(k, _):
        for j in range(fc // 16):
            rows_v[k, pl.ds(16 * j, 16)] = jnp.zeros((16,), jnp.float32)
        return 0

    lax.fori_loop(0, K, zrow, 0)

    def zcp(j, _):
        pltpu.sync_copy(rows_v, acc.at[pl.ds(s * RPW + j * K, K)])
        return 0

    lax.fori_loop(0, RPW // K, zcp, 0)
    plsc.subcore_barrier()


def _writeback(acc, out_h, c, s):
    plsc.subcore_barrier()

    def wb(j, _):
        r0 = s * RPW + j * K
        pltpu.sync_copy(acc.at[pl.ds(r0, K)], out_h.at[pl.ds(c * NP + r0, K)])
        return 0

    lax.fori_loop(0, RPW // K, wb, 0)


GB = 8


def _make_sc_propagate(fc, feat_split):
    rpw = ((EP // NS) if feat_split else (EP // (NC * NS))) // K
    ngroups = rpw // GB

    def body(tbl_h, src_h, dst_h, w_h, out_h,
             idxs_b, idxd_b, w_b, r0_, r1_, acc,
             g0_, g1_, s0_, s1_):
        c = lax.axis_index("c")
        s = lax.axis_index("s")
        base = (s if feat_split else s * NC + c) * rpw
        rows = (r0_, r1_)
        gsem = (g0_, g1_)
        ssem = (s0_, s1_)
        _zero_acc_and_barrier(rows[0], acc, s, fc)

        def group(gi, _):
            r0 = base + gi * GB
            pltpu.sync_copy(src_h.at[pl.ds(r0, GB)], idxs_b)
            pltpu.sync_copy(dst_h.at[pl.ds(r0, GB)], idxd_b)
            pltpu.sync_copy(w_h.at[pl.ds(r0, GB)], w_b)
            if feat_split:
                toff = c * NP

                def adj(j, _):
                    for u in range(K // 16):
                        sl = pl.ds(16 * u, 16)
                        idxs_b[j, sl] = idxs_b[j, sl] + toff
                    return 0

                lax.fori_loop(0, GB, adj, 0)
            gpend = [None] * GB
            gpend[0] = pltpu.async_copy(tbl_h.at[idxs_b.at[0]], rows[0],
                                        gsem[0])
            for j in range(GB):
                b = j % 2
                if j + 1 < GB:
                    gpend[j + 1] = pltpu.async_copy(tbl_h.at[idxs_b.at[j + 1]],
                                                    rows[1 - b], gsem[1 - b])
                gpend[j].wait()

                _j, _b = j, b

                @plsc.parallel_loop(0, K // 16, unroll=2)
                def sgroup(g, _j=_j, _b=_b):
                    wv = w_b[_j, pl.ds(16 * g, 16)]
                    for l in range(16):
                        wk = wv[l]
                        k = 16 * g + l
                        for u in range(fc // 16):
                            sl = pl.ds(16 * u, 16)
                            rows[_b][k, sl] = rows[_b][k, sl] * wk

                pltpu.sync_copy(rows[b], acc.at[idxd_b.at[j]], add=True)
            return 0

        lax.fori_loop(0, ngroups, group, 0)
        _writeback(acc, out_h, c, s)

    return pl.kernel(
        body,
        out_type=jax.ShapeDtypeStruct((2 * NP, fc), jnp.float32),
        mesh=_sc_mesh,
        scratch_types=[
            pltpu.VMEM((GB, K), jnp.int32),
            pltpu.VMEM((GB, K), jnp.int32),
            pltpu.VMEM((GB, K), jnp.float32),
            pltpu.VMEM((K, fc), jnp.float32),
            pltpu.VMEM((K, fc), jnp.float32),
            pltpu.VMEM_SHARED((NP, fc), jnp.float32),
            pltpu.SemaphoreType.DMA,
            pltpu.SemaphoreType.DMA,
            pltpu.SemaphoreType.DMA,
            pltpu.SemaphoreType.DMA,
        ],
    )


def _make_sc_deg():
    fc = 128
    nloc = EP // (NC * NS)
    nchunks = nloc // K

    def body(dst_h, w_h, out_h, idxd_v, w_v, rows_v, acc):
        c = lax.axis_index("c")
        s = lax.axis_index("s")
        base = (s * NC + c) * nloc
        _zero_acc_and_barrier(rows_v, acc, s, fc)
        ones = jnp.full((16,), 1.0, jnp.float32)

        def chunk(i, _):
            eb = base + i * K
            pltpu.sync_copy(dst_h.at[pl.ds(eb, K)], idxd_v)
            pltpu.sync_copy(w_h.at[pl.ds(eb, K)], w_v)

            def sgroup(g, _):
                wv = w_v[pl.ds(16 * g, 16)]
                for l in range(16):
                    rows_v[16 * g + l, pl.ds(0, 16)] = ones * wv[l]
                return 0

            lax.fori_loop(0, K // 16, sgroup, 0)
            pltpu.sync_copy(rows_v, acc.at[idxd_v], add=True)
            return 0

        lax.fori_loop(0, nchunks, chunk, 0)
        _writeback(acc, out_h, c, s)

    return pl.kernel(
        body,
        out_type=jax.ShapeDtypeStruct((2 * NP, fc), jnp.float32),
        mesh=_sc_mesh,
        scratch_types=[
            pltpu.VMEM((K,), jnp.int32),
            pltpu.VMEM((K,), jnp.float32),
            pltpu.VMEM((K, fc), jnp.float32),
            pltpu.VMEM_SHARED((NP, fc), jnp.float32),
        ],
    )


_sc_deg = _make_sc_deg()
_sc_prop128_edge = _make_sc_propagate(128, feat_split=False)
_sc_prop128_feat = _make_sc_propagate(128, feat_split=True)


def kernel(x, edge_index, batch, edge_weight,
           W_in, b_in, W_h1, b_h1, W_h2, b_h2, W_out, b_out):
    n0, e0 = x.shape[0], edge_weight.shape[0]
    pad_idx = jnp.arange(EP - e0, dtype=jnp.int32) % NP
    src = jnp.concatenate([edge_index[0], pad_idx])
    dst = jnp.concatenate([edge_index[1], pad_idx])
    w = jnp.pad(edge_weight, (0, EP - e0))
    x_pad = jnp.pad(x, ((0, NP - n0), (0, 0)))
    batch2 = jnp.pad(batch, (0, NP - n0),
                     constant_values=NG).reshape(GRID, 1, RB)
    W_out_pad = jnp.pad(W_out, ((0, 0), (0, DOP - W_out.shape[1])))
    b_out_pad = jnp.pad(b_out, (0, DOP - b_out.shape[0]))

    src2 = src.reshape(EP // K, K)
    dst2 = dst.reshape(EP // K, K)
    w2 = w.reshape(EP // K, K)

    deg_parts = _sc_deg(dst, w).reshape(2, NP, 128)
    dis_b, g0 = _tc_pre(deg_parts, x_pad)

    acc1 = _sc_prop128_edge(g0, src2, dst2, w2).reshape(2, NP, 128)
    g1 = _tc_layer1(acc1, g0, dis_b, W_in, b_in)

    acc2 = _sc_prop128_feat(g1.reshape(2 * NP, 128), src2, dst2, w2)
    g2 = _tc_mid(acc2.reshape(2, NP, 128), g1, dis_b, W_h1, b_h1)

    acc3 = _sc_prop128_feat(g2.reshape(2 * NP, 128), src2, dst2, w2)
    r3 = _tc_layer3(acc3.reshape(2, NP, 128), g2, dis_b, W_h2, b_h2, W_out_pad)

    acc4 = _sc_prop128_edge(r3, src2, dst2, w2).reshape(2, NP, DOP)
    out = _tc_final(acc4, r3, dis_b, b_out_pad, batch2)

    return out[:, :40]


# speedup vs baseline: 1.3195x; 1.1259x over previous
import functools

import jax
import jax.numpy as jnp
from jax import lax
from jax.experimental import pallas as pl
from jax.experimental.pallas import tpu as pltpu
from jax.experimental.pallas import tpu_sc as plsc

NP = 10240
EP = 327680
RB = 1024
GRID = NP // RB
DOP = 128
NG = 64

_HIGH = lax.Precision.HIGHEST


def _dot(a, b, dims):
    return lax.dot_general(a, b, dims, precision=_HIGH,
                           preferred_element_type=jnp.float32)


def _pre_body(degp_ref, x_ref, dis_ref, g0_ref):
    t = degp_ref[...]
    deg = t[0, :, 0] + t[1, :, 0] + 1.0
    dis = lax.rsqrt(deg)
    dis_ref[...] = jnp.broadcast_to(dis[:, None], (RB, 128))
    g0_ref[...] = x_ref[...] * dis[:, None]


def _layer1_body(acc_ref, g0_ref, dis_ref, w_ref, b_ref, g1_ref):
    dis = dis_ref[:, 0:1]
    p = dis * (acc_ref[0] + acc_ref[1] + g0_ref[...])
    h = jnp.maximum(_dot(p, w_ref[...], (((1,), (0,)), ((), ()))) +
                    b_ref[...], 0.0)
    g = h * dis
    g1_ref[0] = g[:, :128]
    g1_ref[1] = g[:, 128:]


def _mid_body(acc_ref, g_ref, dis_ref, w_ref, b_ref, out_ref):
    dis = dis_ref[:, 0:1]
    p0 = dis * (acc_ref[0] + g_ref[0])
    p1 = dis * (acc_ref[1] + g_ref[1])
    h = (_dot(p0, w_ref[0:128, :], (((1,), (0,)), ((), ()))) +
         _dot(p1, w_ref[128:256, :], (((1,), (0,)), ((), ()))) + b_ref[...])
    h = jnp.maximum(h, 0.0)
    g = h * dis
    out_ref[0] = g[:, :128]
    out_ref[1] = g[:, 128:]


def _layer3_body(acc_ref, g_ref, dis_ref, w_ref, b_ref, wout_ref, r3_ref):
    dis = dis_ref[:, 0:1]
    p0 = dis * (acc_ref[0] + g_ref[0])
    p1 = dis * (acc_ref[1] + g_ref[1])
    h = (_dot(p0, w_ref[0:128, :], (((1,), (0,)), ((), ()))) +
         _dot(p1, w_ref[128:256, :], (((1,), (0,)), ((), ()))) + b_ref[...])
    h = jnp.maximum(h, 0.0)
    q = _dot(h, wout_ref[...], (((1,), (0,)), ((), ())))
    r3_ref[...] = q * dis


def _final_body(acc_ref, r3_ref, dis_ref, bout_ref, batch_ref, out_ref,
                s_acc, c_acc):
    i = pl.program_id(0)

    @pl.when(i == 0)
    def _():
        s_acc[...] = jnp.zeros_like(s_acc)
        c_acc[...] = jnp.zeros_like(c_acc)

    dis = dis_ref[:, 0:1]
    y = dis * (acc_ref[0] + acc_ref[1] + r3_ref[...]) + bout_ref[...]
    bt = batch_ref[0, 0]
    onehot = (bt[:, None] == lax.broadcasted_iota(jnp.int32, (RB, NG), 1)
              ).astype(jnp.float32)
    s_acc[...] += _dot(onehot, y, (((0,), (0,)), ((), ())))
    c_acc[...] += jnp.sum(onehot, axis=0, keepdims=True)

    @pl.when(i == GRID - 1)
    def _():
        pooled = s_acc[...] / jnp.maximum(c_acc[...], 1.0).T
        valid = lax.broadcasted_iota(jnp.int32, (NG, DOP), 1) < 40
        neg = jnp.float32(-1e30)
        masked = jnp.where(valid, pooled, neg)
        m = jnp.max(masked, axis=1, keepdims=True)
        e = jnp.where(valid, jnp.exp(masked - m), 0.0)
        lse = jnp.log(jnp.sum(e, axis=1, keepdims=True))
        out_ref[...] = jnp.where(valid, masked - m - lse, 0.0)


def _tc_pre(deg_parts, x_pad):
    return pl.pallas_call(
        _pre_body,
        grid=(GRID,),
        in_specs=[
            pl.BlockSpec((2, RB, 128), lambda i: (0, i, 0)),
            pl.BlockSpec((RB, 128), lambda i: (i, 0)),
        ],
        out_specs=[
            pl.BlockSpec((RB, 128), lambda i: (i, 0)),
            pl.BlockSpec((RB, 128), lambda i: (i, 0)),
        ],
        out_shape=[
            jax.ShapeDtypeStruct((NP, 128), jnp.float32),
            jax.ShapeDtypeStruct((NP, 128), jnp.float32),
        ],
    )(deg_parts, x_pad)


def _tc_layer1(acc, g0, dis_b, W_in, b_in):
    return pl.pallas_call(
        _layer1_body,
        grid=(GRID,),
        in_specs=[
            pl.BlockSpec((2, RB, 128), lambda i: (0, i, 0)),
            pl.BlockSpec((RB, 128), lambda i: (i, 0)),
            pl.BlockSpec((RB, 128), lambda i: (i, 0)),
            pl.BlockSpec((128, 256), lambda i: (0, 0)),
            pl.BlockSpec((256,), lambda i: (0,)),
        ],
        out_specs=pl.BlockSpec((2, RB, 128), lambda i: (0, i, 0)),
        out_shape=jax.ShapeDtypeStruct((2, NP, 128), jnp.float32),
    )(acc, g0, dis_b, W_in, b_in)


def _tc_mid(acc, g, dis_b, W, b):
    return pl.pallas_call(
        _mid_body,
        grid=(GRID,),
        in_specs=[
            pl.BlockSpec((2, RB, 128), lambda i: (0, i, 0)),
            pl.BlockSpec((2, RB, 128), lambda i: (0, i, 0)),
            pl.BlockSpec((RB, 128), lambda i: (i, 0)),
            pl.BlockSpec((256, 256), lambda i: (0, 0)),
            pl.BlockSpec((256,), lambda i: (0,)),
        ],
        out_specs=pl.BlockSpec((2, RB, 128), lambda i: (0, i, 0)),
        out_shape=jax.ShapeDtypeStruct((2, NP, 128), jnp.float32),
    )(acc, g, dis_b, W, b)


def _tc_layer3(acc, g, dis_b, W, b, W_out_pad):
    return pl.pallas_call(
        _layer3_body,
        grid=(GRID,),
        in_specs=[
            pl.BlockSpec((2, RB, 128), lambda i: (0, i, 0)),
            pl.BlockSpec((2, RB, 128), lambda i: (0, i, 0)),
            pl.BlockSpec((RB, 128), lambda i: (i, 0)),
            pl.BlockSpec((256, 256), lambda i: (0, 0)),
            pl.BlockSpec((256,), lambda i: (0,)),
            pl.BlockSpec((256, DOP), lambda i: (0, 0)),
        ],
        out_specs=pl.BlockSpec((RB, DOP), lambda i: (i, 0)),
        out_shape=jax.ShapeDtypeStruct((NP, DOP), jnp.float32),
    )(acc, g, dis_b, W, b, W_out_pad)


def _tc_final(acc, r3, dis_b, b_out_pad, batch2):
    return pl.pallas_call(
        _final_body,
        grid=(GRID,),
        in_specs=[
            pl.BlockSpec((2, RB, DOP), lambda i: (0, i, 0)),
            pl.BlockSpec((RB, DOP), lambda i: (i, 0)),
            pl.BlockSpec((RB, 128), lambda i: (i, 0)),
            pl.BlockSpec((DOP,), lambda i: (0,)),
            pl.BlockSpec((1, 1, RB), lambda i: (i, 0, 0)),
        ],
        out_specs=pl.BlockSpec((NG, DOP), lambda i: (0, 0)),
        out_shape=jax.ShapeDtypeStruct((NG, DOP), jnp.float32),
        scratch_shapes=[
            pltpu.VMEM((NG, DOP), jnp.float32),
            pltpu.VMEM((1, NG), jnp.float32),
        ],
    )(acc, r3, dis_b, b_out_pad, batch2)


NC, NS = 2, 16
K = 128
RPW = NP // NS

_sc_mesh = plsc.VectorSubcoreMesh(core_axis_name="c", subcore_axis_name="s",
                                  num_cores=NC, num_subcores=NS)


def _zero_acc_and_barrier(rows_v, acc, s, fc):
    def zrow(k, _):
        for j in range(fc // 16):
            rows_v[k, pl.ds(16 * j, 16)] = jnp.zeros((16,), jnp.float32)
        return 0

    lax.fori_loop(0, K, zrow, 0)

    def zcp(j, _):
        pltpu.sync_copy(rows_v, acc.at[pl.ds(s * RPW + j * K, K)])
        return 0

    lax.fori_loop(0, RPW // K, zcp, 0)
    plsc.subcore_barrier()


def _writeback(acc, out_h, c, s):
    plsc.subcore_barrier()

    def wb(j, _):
        r0 = s * RPW + j * K
        pltpu.sync_copy(acc.at[pl.ds(r0, K)], out_h.at[pl.ds(c * NP + r0, K)])
        return 0

    lax.fori_loop(0, RPW // K, wb, 0)


GB = 8


def _make_sc_propagate(fc, feat_split):
    rpw = ((EP // NS) if feat_split else (EP // (NC * NS))) // K
    ngroups = rpw // GB

    def body(tbl_h, src_h, dst_h, w_h, out_h,
             idxs_b, idxd_b, w_b, r0_, r1_, acc,
             g0_, g1_, s0_, s1_, s2_):
        c = lax.axis_index("c")
        s = lax.axis_index("s")
        base = (s if feat_split else s * NC + c) * rpw
        rows = (r0_, r1_)
        gsem = (g0_, g1_)
        ssem = (s0_, s1_, s2_)
        _zero_acc_and_barrier(rows[0], acc, s, fc)

        def group(gi, _):
            r0 = base + gi * GB
            h1 = pltpu.async_copy(src_h.at[pl.ds(r0, GB)], idxs_b, ssem[0])
            h2 = pltpu.async_copy(dst_h.at[pl.ds(r0, GB)], idxd_b, ssem[1])
            h3 = pltpu.async_copy(w_h.at[pl.ds(r0, GB)], w_b, ssem[2])
            h1.wait()
            h2.wait()
            h3.wait()
            if feat_split:
                toff = c * NP

                def adj(j, _):
                    for u in range(K // 16):
                        sl = pl.ds(16 * u, 16)
                        idxs_b[j, sl] = idxs_b[j, sl] + toff
                    return 0

                lax.fori_loop(0, GB, adj, 0)
            gpend = [None] * GB
            gpend[0] = pltpu.async_copy(tbl_h.at[idxs_b.at[0]], rows[0],
                                        gsem[0])
            for j in range(GB):
                b = j % 2
                if j + 1 < GB:
                    gpend[j + 1] = pltpu.async_copy(tbl_h.at[idxs_b.at[j + 1]],
                                                    rows[1 - b], gsem[1 - b])
                gpend[j].wait()

                _j, _b = j, b

                @plsc.parallel_loop(0, K // 16, unroll=2)
                def sgroup(g, _j=_j, _b=_b):
                    wv = w_b[_j, pl.ds(16 * g, 16)]
                    for l in range(16):
                        wk = wv[l]
                        k = 16 * g + l
                        for u in range(fc // 16):
                            sl = pl.ds(16 * u, 16)
                            rows[_b][k, sl] = rows[_b][k, sl] * wk

                pltpu.sync_copy(rows[b], acc.at[idxd_b.at[j]], add=True)
            return 0

        lax.fori_loop(0, ngroups, group, 0)
        _writeback(acc, out_h, c, s)

    return pl.kernel(
        body,
        out_type=jax.ShapeDtypeStruct((2 * NP, fc), jnp.float32),
        mesh=_sc_mesh,
        scratch_types=[
            pltpu.VMEM((GB, K), jnp.int32),
            pltpu.VMEM((GB, K), jnp.int32),
            pltpu.VMEM((GB, K), jnp.float32),
            pltpu.VMEM((K, fc), jnp.float32),
            pltpu.VMEM((K, fc), jnp.float32),
            pltpu.VMEM_SHARED((NP, fc), jnp.float32),
            pltpu.SemaphoreType.DMA,
            pltpu.SemaphoreType.DMA,
            pltpu.SemaphoreType.DMA,
            pltpu.SemaphoreType.DMA,
            pltpu.SemaphoreType.DMA,
        ],
    )


def _make_sc_deg():
    fc = 128
    rpw = (EP // (NC * NS)) // K
    ngroups = rpw // GB

    def body(dst_h, w_h, out_h, idxd_b, w_b, rows_v, acc, s0_, s1_):
        c = lax.axis_index("c")
        s = lax.axis_index("s")
        base = (s * NC + c) * rpw
        _zero_acc_and_barrier(rows_v, acc, s, fc)
        ones = jnp.full((16,), 1.0, jnp.float32)

        def group(gi, _):
            r0 = base + gi * GB
            h1 = pltpu.async_copy(dst_h.at[pl.ds(r0, GB)], idxd_b, s0_)
            h2 = pltpu.async_copy(w_h.at[pl.ds(r0, GB)], w_b, s1_)
            h1.wait()
            h2.wait()
            for j in range(GB):
                _j = j

                @plsc.parallel_loop(0, K // 16, unroll=2)
                def sgroup(g, _j=_j):
                    wv = w_b[_j, pl.ds(16 * g, 16)]
                    for l in range(16):
                        rows_v[16 * g + l, pl.ds(0, 16)] = ones * wv[l]

                pltpu.sync_copy(rows_v, acc.at[idxd_b.at[j]], add=True)
            return 0

        lax.fori_loop(0, ngroups, group, 0)
        _writeback(acc, out_h, c, s)

    return pl.kernel(
        body,
        out_type=jax.ShapeDtypeStruct((2 * NP, fc), jnp.float32),
        mesh=_sc_mesh,
        scratch_types=[
            pltpu.VMEM((GB, K), jnp.int32),
            pltpu.VMEM((GB, K), jnp.float32),
            pltpu.VMEM((K, fc), jnp.float32),
            pltpu.VMEM_SHARED((NP, fc), jnp.float32),
            pltpu.SemaphoreType.DMA,
            pltpu.SemaphoreType.DMA,
        ],
    )


_sc_deg = _make_sc_deg()
_sc_prop128_edge = _make_sc_propagate(128, feat_split=False)
_sc_prop128_feat = _make_sc_propagate(128, feat_split=True)


def kernel(x, edge_index, batch, edge_weight,
           W_in, b_in, W_h1, b_h1, W_h2, b_h2, W_out, b_out):
    n0, e0 = x.shape[0], edge_weight.shape[0]
    pad_idx = jnp.arange(EP - e0, dtype=jnp.int32) % NP
    src = jnp.concatenate([edge_index[0], pad_idx])
    dst = jnp.concatenate([edge_index[1], pad_idx])
    w = jnp.pad(edge_weight, (0, EP - e0))
    x_pad = jnp.pad(x, ((0, NP - n0), (0, 0)))
    batch2 = jnp.pad(batch, (0, NP - n0),
                     constant_values=NG).reshape(GRID, 1, RB)
    W_out_pad = jnp.pad(W_out, ((0, 0), (0, DOP - W_out.shape[1])))
    b_out_pad = jnp.pad(b_out, (0, DOP - b_out.shape[0]))

    src2 = src.reshape(EP // K, K)
    dst2 = dst.reshape(EP // K, K)
    w2 = w.reshape(EP // K, K)

    deg_parts = _sc_deg(dst2, w2).reshape(2, NP, 128)
    dis_b, g0 = _tc_pre(deg_parts, x_pad)

    acc1 = _sc_prop128_edge(g0, src2, dst2, w2).reshape(2, NP, 128)
    g1 = _tc_layer1(acc1, g0, dis_b, W_in, b_in)

    acc2 = _sc_prop128_feat(g1.reshape(2 * NP, 128), src2, dst2, w2)
    g2 = _tc_mid(acc2.reshape(2, NP, 128), g1, dis_b, W_h1, b_h1)

    acc3 = _sc_prop128_feat(g2.reshape(2 * NP, 128), src2, dst2, w2)
    r3 = _tc_layer3(acc3.reshape(2, NP, 128), g2, dis_b, W_h2, b_h2, W_out_pad)

    acc4 = _sc_prop128_edge(r3, src2, dst2, w2).reshape(2, NP, DOP)
    out = _tc_final(acc4, r3, dis_b, b_out_pad, batch2)

    return out[:, :40]
